# baseline (speedup 1.0000x reference)
import functools

import jax
import jax.numpy as jnp
from jax import lax
from jax.experimental import pallas as pl
from jax.experimental.pallas import tpu as pltpu
from jax.experimental.pallas import tpu_sc as plsc

_N = 10000
_D = 128
_E = 320000
_HALF = 5000
_ACC = 5120
_NC = 10016
_NW = 32
_EW = 10240
_EPAD = _NW * _EW
_SLOT = _EW
_BN = 400


def _pre_body(x_ref, pos_ref, len_ref, wa_ref, wpl_ref, b_ref, out_ref):
    pos = pos_ref[...].astype(jnp.float32) * (1.0 / 50.0)
    ln = len_ref[...].astype(jnp.float32) * (1.0 / 500.0)
    acc = jnp.dot(x_ref[...], wa_ref[...], preferred_element_type=jnp.float32)
    acc = acc + pos * wpl_ref[0:1, :] + ln * wpl_ref[1:2, :] + b_ref[...]
    out_ref[...] = acc


def _pre(x, pos, ln, wa, wpl, b):
    return pl.pallas_call(
        _pre_body,
        grid=(_N // _BN,),
        in_specs=[
            pl.BlockSpec((_BN, _D), lambda i: (i, 0)),
            pl.BlockSpec((_BN, 1), lambda i: (i, 0)),
            pl.BlockSpec((_BN, 1), lambda i: (i, 0)),
            pl.BlockSpec((_D, _D), lambda i: (0, 0)),
            pl.BlockSpec((2, _D), lambda i: (0, 0)),
            pl.BlockSpec((1, _D), lambda i: (0, 0)),
        ],
        out_specs=pl.BlockSpec((_BN, _D), lambda i: (i, 0)),
        out_shape=jax.ShapeDtypeStruct((_N, _D), jnp.float32),
    )(x, pos, ln, wa, wpl, b)


def _sc_partition(srcE, dstE):
    mesh = plsc.VectorSubcoreMesh(
        core_axis_name="c", subcore_axis_name="s", num_cores=2, num_subcores=16
    )

    @functools.partial(
        pl.kernel,
        mesh=mesh,
        out_type=[
            jax.ShapeDtypeStruct((_NW, 2, _SLOT), jnp.int32),
            jax.ShapeDtypeStruct((_NW, 2, _SLOT), jnp.int32),
            jax.ShapeDtypeStruct((_NW, 2, 16), jnp.int32),
            jax.ShapeDtypeStruct((_NW, _NC), jnp.float32),
        ],
        scratch_types=[
            pltpu.VMEM((4, 128), jnp.int32),
            pltpu.VMEM((4, 128), jnp.int32),
            pltpu.VMEM((_SLOT + 16,), jnp.int32),
            pltpu.VMEM((_SLOT + 16,), jnp.int32),
            pltpu.VMEM((_SLOT + 16,), jnp.int32),
            pltpu.VMEM((_SLOT + 16,), jnp.int32),
            pltpu.VMEM((_NC,), jnp.float32),
            pltpu.VMEM((2, 16), jnp.int32),
        ],
        compiler_params=pltpu.CompilerParams(needs_layout_passes=False),
    )
    def k(srcE_hbm, dstE_hbm, psrc_hbm, pdoff_hbm, counts_hbm, cnt_hbm,
          es_v, ed_v, b0s, b0d, b1s, b1d, cnt_v, cout_v):
        c = lax.axis_index("c")
        s = lax.axis_index("s")
        wid = c * 16 + s

        ziv = jnp.zeros((16,), jnp.int32)
        zfv = jnp.zeros((16,), jnp.float32)
        ones = jnp.full((16,), 1.0, jnp.float32)
        dummy = ziv + (5008 + lax.div(wid, 2))

        def pre(i, carry):
            b0s[pl.ds(i * 16, 16)] = ziv
            b0d[pl.ds(i * 16, 16)] = dummy
            b1s[pl.ds(i * 16, 16)] = ziv
            b1d[pl.ds(i * 16, 16)] = dummy
            return carry

        lax.fori_loop(0, (_SLOT + 16) // 16, pre, 0)

        def zcnt(i, carry):
            cnt_v[pl.ds(i * 16, 16)] = zfv
            return carry

        lax.fori_loop(0, _NC // 16, zcnt, 0)

        def blk(bb, carry):
            off0, off1 = carry
            pltpu.sync_copy(srcE_hbm.at[wid, bb], es_v)
            pltpu.sync_copy(dstE_hbm.at[wid, bb], ed_v)
            for g in range(32):
                r, co = g // 8, (g % 8) * 16
                s16 = es_v[r, pl.ds(co, 16)]
                d16 = ed_v[r, pl.ds(co, 16)]
                h0 = d16 < _HALF
                h1 = jnp.logical_and(d16 >= _HALF, d16 < _N)
                plsc.store_compressed(b0s.at[pl.ds(off0, 16)], s16, mask=h0)
                plsc.store_compressed(b0d.at[pl.ds(off0, 16)], d16, mask=h0)
                off0 = off0 + jnp.max(plsc.all_reduce_population_count(h0))
                plsc.store_compressed(b1s.at[pl.ds(off1, 16)], s16, mask=h1)
                plsc.store_compressed(b1d.at[pl.ds(off1, 16)], d16 - _HALF,
                                      mask=h1)
                off1 = off1 + jnp.max(plsc.all_reduce_population_count(h1))
                plsc.addupdate_scatter(cnt_v, [d16], ones)
            return (off0, off1)

        off0, off1 = lax.fori_loop(
            0, _EW // 512, blk, (jnp.int32(0), jnp.int32(0)))

        pltpu.sync_copy(b0s.at[pl.ds(0, _SLOT)], psrc_hbm.at[wid, 0])
        pltpu.sync_copy(b1s.at[pl.ds(0, _SLOT)], psrc_hbm.at[wid, 1])
        pltpu.sync_copy(b0d.at[pl.ds(0, _SLOT)], pdoff_hbm.at[wid, 0])
        pltpu.sync_copy(b1d.at[pl.ds(0, _SLOT)], pdoff_hbm.at[wid, 1])
        cout_v[0, pl.ds(0, 16)] = ziv + off0
        cout_v[1, pl.ds(0, 16)] = ziv + off1
        pltpu.sync_copy(cout_v, counts_hbm.at[wid])
        pltpu.sync_copy(cnt_v, cnt_hbm.at[wid])

    return k(srcE, dstE)


def _sc_agg(x0, psrc5, pdoff5, counts):
    mesh = plsc.VectorSubcoreMesh(
        core_axis_name="c", subcore_axis_name="s", num_cores=2, num_subcores=16
    )

    @functools.partial(
        pl.kernel,
        mesh=mesh,
        out_type=jax.ShapeDtypeStruct((2, _ACC, _D), jnp.float32),
        scratch_types=[
            pltpu.VMEM((4, 128), jnp.int32),
            pltpu.VMEM((4, 128), jnp.int32),
            pltpu.VMEM((32, _D), jnp.float32),
            pltpu.VMEM((32, _D), jnp.float32),
            pltpu.VMEM((16,), jnp.int32),
            pltpu.VMEM_SHARED((_N, _D), jnp.float32),
            pltpu.VMEM_SHARED((_ACC, _D), jnp.float32),
            pltpu.SemaphoreType.DMA,
            pltpu.SemaphoreType.DMA,
        ],
        compiler_params=pltpu.CompilerParams(needs_layout_passes=False),
    )
    def k(x0_hbm, psrc_hbm, pdoff_hbm, counts_hbm, agg_hbm,
          is_v, id_v, rows0_v, rows1_v, cnt_v, x0_s, acc_s, gsem, ssem):
        c = lax.axis_index("c")
        s = lax.axis_index("s")

        zfv = jnp.zeros((16,), jnp.float32)

        def zrow(i, carry):
            rows0_v[i, pl.ds(0, 16)] = zfv
            rows0_v[i, pl.ds(16, 16)] = zfv
            rows0_v[i, pl.ds(32, 16)] = zfv
            rows0_v[i, pl.ds(48, 16)] = zfv
            rows0_v[i, pl.ds(64, 16)] = zfv
            rows0_v[i, pl.ds(80, 16)] = zfv
            rows0_v[i, pl.ds(96, 16)] = zfv
            rows0_v[i, pl.ds(112, 16)] = zfv
            return carry

        lax.fori_loop(0, 32, zrow, 0)

        for t in range(10):
            pltpu.sync_copy(rows0_v, acc_s.at[pl.ds(s * 320 + t * 32, 32)])

        for kk in range(4):
            b = s + 16 * kk
            pltpu.sync_copy(x0_hbm.at[pl.ds(b * 128, 128)],
                            x0_s.at[pl.ds(b * 128, 128)])

        @pl.when(s <= 13)
        def _():
            b = s + 64
            pltpu.sync_copy(x0_hbm.at[pl.ds(b * 128, 128)],
                            x0_s.at[pl.ds(b * 128, 128)])

        @pl.when(s == 15)
        def _():
            pltpu.sync_copy(x0_hbm.at[pl.ds(9984, 16)],
                            x0_s.at[pl.ds(9984, 16)])

        plsc.subcore_barrier()

        bufs = (rows0_v, rows1_v)

        for u in range(2):
            w = 2 * s + u
            pltpu.sync_copy(counts_hbm.at[w, c], cnt_v)
            n = jnp.max(cnt_v[...])
            nb = lax.div(n + 511, 512)

            def blk(bb, carry):
                pltpu.sync_copy(psrc_hbm.at[w, c, bb], is_v)
                pltpu.sync_copy(pdoff_hbm.at[w, c, bb], id_v)

                def gfire(j, buf):
                    return pltpu.async_copy(
                        x0_s.at[is_v.at[j // 4, pl.ds((j % 4) * 32, 32)]],
                        buf, gsem)

                g = gfire(0, bufs[0])
                sprev = None
                for j in range(16):
                    cur = bufs[j % 2]
                    nxt = bufs[(j + 1) % 2]
                    g.wait()
                    if sprev is not None:
                        sprev.wait()
                    if j < 15:
                        g = gfire(j + 1, nxt)
                    sprev = pltpu.async_copy(
                        cur,
                        acc_s.at[id_v.at[j // 4, pl.ds((j % 4) * 32, 32)]],
                        ssem, add=True)
                sprev.wait()
                return carry

            lax.fori_loop(0, nb, blk, 0)

        plsc.subcore_barrier()
        pltpu.sync_copy(acc_s.at[pl.ds(s * 320, 320)],
                        agg_hbm.at[c, pl.ds(s * 320, 320)])

    return k(x0, psrc5, pdoff5, counts)


def _post_body(x0_ref, agg_ref, cnt_ref, rr_ref, wl_ref, bl_ref, wr_ref,
               wres_ref, bres_ref, wsc_ref, scal_ref, out_ref):
    cnt = jnp.sum(cnt_ref[...], axis=1, keepdims=True)
    aggm = agg_ref[...] * (1.0 / jnp.maximum(cnt, 1.0))
    x0 = x0_ref[...]
    h = jax.nn.relu(
        jnp.dot(aggm, wl_ref[...], preferred_element_type=jnp.float32)
        + bl_ref[...]
        + jnp.dot(x0, wr_ref[...], preferred_element_type=jnp.float32)
    )
    h = h + jnp.dot(x0, wres_ref[...], preferred_element_type=jnp.float32) + bres_ref[...]
    gnn = jnp.sum(h * wsc_ref[...], axis=1, keepdims=True)
    b_sc = scal_ref[0, 0]
    a = jax.nn.sigmoid(scal_ref[0, 1])
    out_ref[...] = a * rr_ref[...] + (1.0 - a) * (gnn + b_sc)


def _post(x0, agg, cntT, rr, wl, bl, wr, wres, bres, wsc, scal):
    return pl.pallas_call(
        _post_body,
        grid=(_N // _BN,),
        in_specs=[
            pl.BlockSpec((_BN, _D), lambda i: (i, 0)),
            pl.BlockSpec((_BN, _D), lambda i: (i, 0)),
            pl.BlockSpec((_BN, _NW), lambda i: (i, 0)),
            pl.BlockSpec((_BN, 1), lambda i: (i, 0)),
            pl.BlockSpec((_D, _D), lambda i: (0, 0)),
            pl.BlockSpec((1, _D), lambda i: (0, 0)),
            pl.BlockSpec((_D, _D), lambda i: (0, 0)),
            pl.BlockSpec((_D, _D), lambda i: (0, 0)),
            pl.BlockSpec((1, _D), lambda i: (0, 0)),
            pl.BlockSpec((1, _D), lambda i: (0, 0)),
            pl.BlockSpec(memory_space=pltpu.SMEM),
        ],
        out_specs=pl.BlockSpec((_BN, 1), lambda i: (i, 0)),
        out_shape=jax.ShapeDtypeStruct((_N, 1), jnp.float32),
    )(x0, agg, cntT, rr, wl, bl, wr, wres, bres, wsc, scal)


def kernel(x, edge_index, reranker_scores, positions, lengths, W_fp, b_fp,
           W_l, b_l, W_r, W_res, b_res, W_sc, b_sc, alpha):
    x0 = _pre(
        x,
        positions.reshape(_N, 1),
        lengths.reshape(_N, 1),
        W_fp[:_D],
        W_fp[_D:_D + 2],
        b_fp.reshape(1, _D),
    )

    src, dst = edge_index[0], edge_index[1]
    pad = _EPAD - _E
    srcE = jnp.concatenate([src, jnp.zeros((pad,), jnp.int32)]
                           ).reshape(_NW, _EW // 512, 4, 128)
    dstE = jnp.concatenate([dst, jnp.full((pad,), _N, jnp.int32)]
                           ).reshape(_NW, _EW // 512, 4, 128)

    psrc, pdoff, counts, cnt = _sc_partition(srcE, dstE)

    aggh = _sc_agg(
        x0,
        psrc.reshape(_NW, 2, _SLOT // 512, 4, 128),
        pdoff.reshape(_NW, 2, _SLOT // 512, 4, 128),
        counts,
    )
    agg = jnp.concatenate([aggh[0, :_HALF], aggh[1, :_HALF]], axis=0)

    scal = jnp.stack([b_sc[0], alpha]).reshape(1, 2)
    out = _post(
        x0,
        agg,
        cnt[:, :_N].T,
        reranker_scores.reshape(_N, 1),
        W_l,
        b_l.reshape(1, _D),
        W_r,
        W_res,
        b_res.reshape(1, _D),
        W_sc.reshape(1, _D),
        scal,
    )
    return out.reshape(_N)


# speedup vs baseline: 1.0433x; 1.0433x over previous
import functools

import jax
import jax.numpy as jnp
from jax import lax
from jax.experimental import pallas as pl
from jax.experimental.pallas import tpu as pltpu
from jax.experimental.pallas import tpu_sc as plsc

_N = 10000
_D = 128
_E = 320000
_HALF = 5000
_ACC = 5120
_NC = 10016
_NW = 32
_EW = 10240
_EPAD = _NW * _EW
_SLOT = _EW
_BN = 400


def _pre_body(x_ref, pos_ref, len_ref, wa_ref, wpl_ref, b_ref,
              bl_ref, wr_ref, wres_ref, bres_ref, x0_ref, xr_ref, hres_ref):
    pos = pos_ref[...].astype(jnp.float32) * (1.0 / 50.0)
    ln = len_ref[...].astype(jnp.float32) * (1.0 / 500.0)
    x0 = jnp.dot(x_ref[...], wa_ref[...], preferred_element_type=jnp.float32)
    x0 = x0 + pos * wpl_ref[0:1, :] + ln * wpl_ref[1:2, :] + b_ref[...]
    x0_ref[...] = x0
    xr_ref[...] = (
        jnp.dot(x0, wr_ref[...], preferred_element_type=jnp.float32)
        + bl_ref[...])
    hres_ref[...] = (
        jnp.dot(x0, wres_ref[...], preferred_element_type=jnp.float32)
        + bres_ref[...])


def _pre(x, pos, ln, wa, wpl, b, bl, wr, wres, bres):
    return pl.pallas_call(
        _pre_body,
        grid=(_N // _BN,),
        in_specs=[
            pl.BlockSpec((_BN, _D), lambda i: (i, 0)),
            pl.BlockSpec((_BN, 1), lambda i: (i, 0)),
            pl.BlockSpec((_BN, 1), lambda i: (i, 0)),
            pl.BlockSpec((_D, _D), lambda i: (0, 0)),
            pl.BlockSpec((2, _D), lambda i: (0, 0)),
            pl.BlockSpec((1, _D), lambda i: (0, 0)),
            pl.BlockSpec((1, _D), lambda i: (0, 0)),
            pl.BlockSpec((_D, _D), lambda i: (0, 0)),
            pl.BlockSpec((_D, _D), lambda i: (0, 0)),
            pl.BlockSpec((1, _D), lambda i: (0, 0)),
        ],
        out_specs=[
            pl.BlockSpec((_BN, _D), lambda i: (i, 0)),
            pl.BlockSpec((_BN, _D), lambda i: (i, 0)),
            pl.BlockSpec((_BN, _D), lambda i: (i, 0)),
        ],
        out_shape=[
            jax.ShapeDtypeStruct((_N, _D), jnp.float32),
            jax.ShapeDtypeStruct((_N, _D), jnp.float32),
            jax.ShapeDtypeStruct((_N, _D), jnp.float32),
        ],
    )(x, pos, ln, wa, wpl, b, bl, wr, wres, bres)


def _sc_partition(srcE, dstE):
    mesh = plsc.VectorSubcoreMesh(
        core_axis_name="c", subcore_axis_name="s", num_cores=2, num_subcores=16
    )

    @functools.partial(
        pl.kernel,
        mesh=mesh,
        out_type=[
            jax.ShapeDtypeStruct((_NW, 2, _SLOT), jnp.int32),
            jax.ShapeDtypeStruct((_NW, 2, _SLOT), jnp.int32),
            jax.ShapeDtypeStruct((_NW, 2, 16), jnp.int32),
            jax.ShapeDtypeStruct((_NW, _NC), jnp.float32),
        ],
        scratch_types=[
            pltpu.VMEM((4, 128), jnp.int32),
            pltpu.VMEM((4, 128), jnp.int32),
            pltpu.VMEM((_SLOT + 16,), jnp.int32),
            pltpu.VMEM((_SLOT + 16,), jnp.int32),
            pltpu.VMEM((_SLOT + 16,), jnp.int32),
            pltpu.VMEM((_SLOT + 16,), jnp.int32),
            pltpu.VMEM((_NC,), jnp.float32),
            pltpu.VMEM((2, 16), jnp.int32),
        ],
        compiler_params=pltpu.CompilerParams(needs_layout_passes=False),
    )
    def k(srcE_hbm, dstE_hbm, psrc_hbm, pdoff_hbm, counts_hbm, cnt_hbm,
          es_v, ed_v, b0s, b0d, b1s, b1d, cnt_v, cout_v):
        c = lax.axis_index("c")
        s = lax.axis_index("s")
        wid = c * 16 + s
        nblk = jnp.where(wid < 17, 20, 19)
        start = wid * 19 + jnp.minimum(wid, 17)

        ziv = jnp.zeros((16,), jnp.int32)
        zfv = jnp.zeros((16,), jnp.float32)
        ones = jnp.full((16,), 1.0, jnp.float32)
        dummy = ziv + (5008 + lax.div(wid, 2))

        def pre(i, carry):
            b0s[pl.ds(i * 16, 16)] = ziv
            b0d[pl.ds(i * 16, 16)] = dummy
            b1s[pl.ds(i * 16, 16)] = ziv
            b1d[pl.ds(i * 16, 16)] = dummy
            return carry

        lax.fori_loop(0, (_SLOT + 16) // 16, pre, 0)

        def zcnt(i, carry):
            cnt_v[pl.ds(i * 16, 16)] = zfv
            return carry

        lax.fori_loop(0, _NC // 16, zcnt, 0)

        def blk(bb, carry):
            off0, off1 = carry
            pltpu.sync_copy(srcE_hbm.at[start + bb], es_v)
            pltpu.sync_copy(dstE_hbm.at[start + bb], ed_v)
            for g in range(32):
                r, co = g // 8, (g % 8) * 16
                s16 = es_v[r, pl.ds(co, 16)]
                d16 = ed_v[r, pl.ds(co, 16)]
                h0 = d16 < _HALF
                h1 = d16 >= _HALF
                plsc.store_compressed(b0s.at[pl.ds(off0, 16)], s16, mask=h0)
                plsc.store_compressed(b0d.at[pl.ds(off0, 16)], d16, mask=h0)
                off0 = off0 + plsc.all_reduce_population_count(h0)[0]
                plsc.store_compressed(b1s.at[pl.ds(off1, 16)], s16, mask=h1)
                plsc.store_compressed(b1d.at[pl.ds(off1, 16)], d16 - _HALF,
                                      mask=h1)
                off1 = off1 + plsc.all_reduce_population_count(h1)[0]
                plsc.addupdate_scatter(cnt_v, [d16], ones)
            return (off0, off1)

        off0, off1 = lax.fori_loop(
            0, nblk, blk, (jnp.int32(0), jnp.int32(0)))

        pltpu.sync_copy(b0s.at[pl.ds(0, _SLOT)], psrc_hbm.at[wid, 0])
        pltpu.sync_copy(b1s.at[pl.ds(0, _SLOT)], psrc_hbm.at[wid, 1])
        pltpu.sync_copy(b0d.at[pl.ds(0, _SLOT)], pdoff_hbm.at[wid, 0])
        pltpu.sync_copy(b1d.at[pl.ds(0, _SLOT)], pdoff_hbm.at[wid, 1])
        cout_v[0, pl.ds(0, 16)] = ziv + off0
        cout_v[1, pl.ds(0, 16)] = ziv + off1
        pltpu.sync_copy(cout_v, counts_hbm.at[wid])
        pltpu.sync_copy(cnt_v, cnt_hbm.at[wid])

    return k(srcE, dstE)


def _sc_agg(x0, psrc5, pdoff5, counts):
    mesh = plsc.VectorSubcoreMesh(
        core_axis_name="c", subcore_axis_name="s", num_cores=2, num_subcores=16
    )

    @functools.partial(
        pl.kernel,
        mesh=mesh,
        out_type=jax.ShapeDtypeStruct((2, _ACC, _D), jnp.float32),
        scratch_types=[
            pltpu.VMEM((4, 128), jnp.int32),
            pltpu.VMEM((4, 128), jnp.int32),
            pltpu.VMEM((32, _D), jnp.float32),
            pltpu.VMEM((32, _D), jnp.float32),
            pltpu.VMEM((16,), jnp.int32),
            pltpu.VMEM_SHARED((_N, _D), jnp.float32),
            pltpu.VMEM_SHARED((_ACC, _D), jnp.float32),
            pltpu.SemaphoreType.DMA,
            pltpu.SemaphoreType.DMA,
        ],
        compiler_params=pltpu.CompilerParams(needs_layout_passes=False),
    )
    def k(x0_hbm, psrc_hbm, pdoff_hbm, counts_hbm, agg_hbm,
          is_v, id_v, rows0_v, rows1_v, cnt_v, x0_s, acc_s, gsem, ssem):
        c = lax.axis_index("c")
        s = lax.axis_index("s")

        zfv = jnp.zeros((16,), jnp.float32)

        def zrow(i, carry):
            rows0_v[i, pl.ds(0, 16)] = zfv
            rows0_v[i, pl.ds(16, 16)] = zfv
            rows0_v[i, pl.ds(32, 16)] = zfv
            rows0_v[i, pl.ds(48, 16)] = zfv
            rows0_v[i, pl.ds(64, 16)] = zfv
            rows0_v[i, pl.ds(80, 16)] = zfv
            rows0_v[i, pl.ds(96, 16)] = zfv
            rows0_v[i, pl.ds(112, 16)] = zfv
            return carry

        lax.fori_loop(0, 32, zrow, 0)

        for t in range(10):
            pltpu.sync_copy(rows0_v, acc_s.at[pl.ds(s * 320 + t * 32, 32)])

        for kk in range(4):
            b = s + 16 * kk
            pltpu.sync_copy(x0_hbm.at[pl.ds(b * 128, 128)],
                            x0_s.at[pl.ds(b * 128, 128)])

        @pl.when(s <= 13)
        def _():
            b = s + 64
            pltpu.sync_copy(x0_hbm.at[pl.ds(b * 128, 128)],
                            x0_s.at[pl.ds(b * 128, 128)])

        @pl.when(s == 15)
        def _():
            pltpu.sync_copy(x0_hbm.at[pl.ds(9984, 16)],
                            x0_s.at[pl.ds(9984, 16)])

        plsc.subcore_barrier()

        bufs = (rows0_v, rows1_v)

        for u in range(2):
            w = 2 * s + u
            pltpu.sync_copy(counts_hbm.at[w, c], cnt_v)
            n = cnt_v[...][0]
            nb = lax.div(n + 511, 512)

            def blk(bb, carry):
                pltpu.sync_copy(psrc_hbm.at[w, c, bb], is_v)
                pltpu.sync_copy(pdoff_hbm.at[w, c, bb], id_v)

                def gfire(j, buf):
                    return pltpu.async_copy(
                        x0_s.at[is_v.at[j // 4, pl.ds((j % 4) * 32, 32)]],
                        buf, gsem)

                g = gfire(0, bufs[0])
                sprev = None
                for j in range(16):
                    cur = bufs[j % 2]
                    nxt = bufs[(j + 1) % 2]
                    g.wait()
                    if sprev is not None:
                        sprev.wait()
                    if j < 15:
                        g = gfire(j + 1, nxt)
                    sprev = pltpu.async_copy(
                        cur,
                        acc_s.at[id_v.at[j // 4, pl.ds((j % 4) * 32, 32)]],
                        ssem, add=True)
                sprev.wait()
                return carry

            lax.fori_loop(0, nb, blk, 0)

        plsc.subcore_barrier()
        pltpu.sync_copy(acc_s.at[pl.ds(s * 320, 320)],
                        agg_hbm.at[c, pl.ds(s * 320, 320)])

    return k(x0, psrc5, pdoff5, counts)


def _post_body(agg_ref, cnt_ref, rr_ref, xr_ref, hres_ref, wl_ref,
               wsc_ref, scal_ref, out_ref):
    cnt = jnp.sum(cnt_ref[...], axis=1, keepdims=True)
    aggm = agg_ref[...] * (1.0 / jnp.maximum(cnt, 1.0))
    h = jax.nn.relu(
        jnp.dot(aggm, wl_ref[...], preferred_element_type=jnp.float32)
        + xr_ref[...]
    )
    h = h + hres_ref[...]
    gnn = jnp.sum(h * wsc_ref[...], axis=1, keepdims=True)
    b_sc = scal_ref[0, 0]
    a = jax.nn.sigmoid(scal_ref[0, 1])
    out_ref[...] = a * rr_ref[...] + (1.0 - a) * (gnn + b_sc)


def _post(agg, cntT, rr, xr, hres, wl, wsc, scal):
    return pl.pallas_call(
        _post_body,
        grid=(_N // _BN,),
        in_specs=[
            pl.BlockSpec((_BN, _D), lambda i: (i, 0)),
            pl.BlockSpec((_BN, _NW), lambda i: (i, 0)),
            pl.BlockSpec((_BN, 1), lambda i: (i, 0)),
            pl.BlockSpec((_BN, _D), lambda i: (i, 0)),
            pl.BlockSpec((_BN, _D), lambda i: (i, 0)),
            pl.BlockSpec((_D, _D), lambda i: (0, 0)),
            pl.BlockSpec((1, _D), lambda i: (0, 0)),
            pl.BlockSpec(memory_space=pltpu.SMEM),
        ],
        out_specs=pl.BlockSpec((_BN, 1), lambda i: (i, 0)),
        out_shape=jax.ShapeDtypeStruct((_N, 1), jnp.float32),
    )(agg, cntT, rr, xr, hres, wl, wsc, scal)


def kernel(x, edge_index, reranker_scores, positions, lengths, W_fp, b_fp,
           W_l, b_l, W_r, W_res, b_res, W_sc, b_sc, alpha):
    x0, xr, hres = _pre(
        x,
        positions.reshape(_N, 1),
        lengths.reshape(_N, 1),
        W_fp[:_D],
        W_fp[_D:_D + 2],
        b_fp.reshape(1, _D),
        b_l.reshape(1, _D),
        W_r,
        W_res,
        b_res.reshape(1, _D),
    )

    srcE = edge_index[0].reshape(_E // 512, 4, 128)
    dstE = edge_index[1].reshape(_E // 512, 4, 128)

    psrc, pdoff, counts, cnt = _sc_partition(srcE, dstE)

    aggh = _sc_agg(
        x0,
        psrc.reshape(_NW, 2, _SLOT // 512, 4, 128),
        pdoff.reshape(_NW, 2, _SLOT // 512, 4, 128),
        counts,
    )
    agg = jnp.concatenate([aggh[0, :_HALF], aggh[1, :_HALF]], axis=0)

    scal = jnp.stack([b_sc[0], alpha]).reshape(1, 2)
    out = _post(
        agg,
        cnt[:, :_N].T,
        reranker_scores.reshape(_N, 1),
        xr,
        hres,
        W_l,
        W_sc.reshape(1, _D),
        scal,
    )
    return out.reshape(_N)


# speedup vs baseline: 1.0586x; 1.0147x over previous
import functools

import jax
import jax.numpy as jnp
from jax import lax
from jax.experimental import pallas as pl
from jax.experimental.pallas import tpu as pltpu
from jax.experimental.pallas import tpu_sc as plsc

_N = 10000
_D = 128
_E = 320000
_HALF = 5000
_ACC = 5120
_NC = 10016
_NW = 32
_EW = 10240
_EPAD = _NW * _EW
_SLOT = _EW
_BN = 400


def _pre_body(x_ref, pos_ref, len_ref, wa_ref, wpl_ref, b_ref,
              bl_ref, wr_ref, wres_ref, bres_ref, x0_ref, xr_ref, hres_ref):
    pos = pos_ref[...].astype(jnp.float32) * (1.0 / 50.0)
    ln = len_ref[...].astype(jnp.float32) * (1.0 / 500.0)
    x0 = jnp.dot(x_ref[...], wa_ref[...], preferred_element_type=jnp.float32)
    x0 = x0 + pos * wpl_ref[0:1, :] + ln * wpl_ref[1:2, :] + b_ref[...]
    x0_ref[...] = x0
    xr_ref[...] = (
        jnp.dot(x0, wr_ref[...], preferred_element_type=jnp.float32)
        + bl_ref[...])
    hres_ref[...] = (
        jnp.dot(x0, wres_ref[...], preferred_element_type=jnp.float32)
        + bres_ref[...])


def _pre(x, pos, ln, wa, wpl, b, bl, wr, wres, bres):
    return pl.pallas_call(
        _pre_body,
        grid=(_N // _BN,),
        in_specs=[
            pl.BlockSpec((_BN, _D), lambda i: (i, 0)),
            pl.BlockSpec((_BN, 1), lambda i: (i, 0)),
            pl.BlockSpec((_BN, 1), lambda i: (i, 0)),
            pl.BlockSpec((_D, _D), lambda i: (0, 0)),
            pl.BlockSpec((2, _D), lambda i: (0, 0)),
            pl.BlockSpec((1, _D), lambda i: (0, 0)),
            pl.BlockSpec((1, _D), lambda i: (0, 0)),
            pl.BlockSpec((_D, _D), lambda i: (0, 0)),
            pl.BlockSpec((_D, _D), lambda i: (0, 0)),
            pl.BlockSpec((1, _D), lambda i: (0, 0)),
        ],
        out_specs=[
            pl.BlockSpec((_BN, _D), lambda i: (i, 0)),
            pl.BlockSpec((_BN, _D), lambda i: (i, 0)),
            pl.BlockSpec((_BN, _D), lambda i: (i, 0)),
        ],
        out_shape=[
            jax.ShapeDtypeStruct((_N, _D), jnp.float32),
            jax.ShapeDtypeStruct((_N, _D), jnp.float32),
            jax.ShapeDtypeStruct((_N, _D), jnp.float32),
        ],
    )(x, pos, ln, wa, wpl, b, bl, wr, wres, bres)


def _sc_partition(srcE, dstE):
    mesh = plsc.VectorSubcoreMesh(
        core_axis_name="c", subcore_axis_name="s", num_cores=2, num_subcores=16
    )

    @functools.partial(
        pl.kernel,
        mesh=mesh,
        out_type=[
            jax.ShapeDtypeStruct((_NW, 2, _SLOT), jnp.int32),
            jax.ShapeDtypeStruct((_NW, 2, _SLOT), jnp.int32),
            jax.ShapeDtypeStruct((_NW, 2, 16), jnp.int32),
            jax.ShapeDtypeStruct((_NW, _NC), jnp.float32),
        ],
        scratch_types=[
            pltpu.VMEM((4, 128), jnp.int32),
            pltpu.VMEM((4, 128), jnp.int32),
            pltpu.VMEM((4, 128), jnp.int32),
            pltpu.VMEM((4, 128), jnp.int32),
            pltpu.VMEM((_SLOT + 16,), jnp.int32),
            pltpu.VMEM((_SLOT + 16,), jnp.int32),
            pltpu.VMEM((_SLOT + 16,), jnp.int32),
            pltpu.VMEM((_SLOT + 16,), jnp.int32),
            pltpu.VMEM((_NC,), jnp.float32),
            pltpu.VMEM((2, 16), jnp.int32),
            pltpu.SemaphoreType.DMA,
            pltpu.SemaphoreType.DMA,
        ],
        compiler_params=pltpu.CompilerParams(needs_layout_passes=False),
    )
    def k(srcE_hbm, dstE_hbm, psrc_hbm, pdoff_hbm, counts_hbm, cnt_hbm,
          es0, ed0, es1, ed1, b0s, b0d, b1s, b1d, cnt_v, cout_v,
          stsem, osem):
        c = lax.axis_index("c")
        s = lax.axis_index("s")
        wid = c * 16 + s
        nblk = jnp.where(wid < 17, 20, 19)
        start = wid * 19 + jnp.minimum(wid, 17)

        ziv = jnp.zeros((16,), jnp.int32)
        zfv = jnp.zeros((16,), jnp.float32)
        ones = jnp.full((16,), 1.0, jnp.float32)
        dummy = ziv + (5008 + lax.div(wid, 2))

        def pre(i, carry):
            b0s[pl.ds(i * 16, 16)] = ziv
            b0d[pl.ds(i * 16, 16)] = dummy
            b1s[pl.ds(i * 16, 16)] = ziv
            b1d[pl.ds(i * 16, 16)] = dummy
            return carry

        lax.fori_loop(0, (_SLOT + 16) // 16, pre, 0)

        def zcnt(i, carry):
            cnt_v[pl.ds(i * 16, 16)] = zfv
            return carry

        lax.fori_loop(0, _NC // 16, zcnt, 0)

        nlast = _E // 512 - 1

        def stage(bidx, sbuf, dbuf):
            bi = jnp.minimum(start + bidx, nlast)
            pltpu.async_copy(srcE_hbm.at[bi], sbuf, stsem)
            pltpu.async_copy(dstE_hbm.at[bi], dbuf, stsem)

        def wait_stage(sbuf, dbuf):
            pltpu.make_async_copy(srcE_hbm.at[0], sbuf, stsem).wait()
            pltpu.make_async_copy(dstE_hbm.at[0], dbuf, stsem).wait()

        def process(bb, es_v, ed_v, off0, off1):
            vm = (ziv + bb) < (ziv + nblk)
            for g in range(32):
                r, co = g // 8, (g % 8) * 16
                s16 = es_v[r, pl.ds(co, 16)]
                d16 = ed_v[r, pl.ds(co, 16)]
                h0 = jnp.logical_and(d16 < _HALF, vm)
                h1 = jnp.logical_and(d16 >= _HALF, vm)
                plsc.store_compressed(b0s.at[pl.ds(off0, 16)], s16, mask=h0)
                plsc.store_compressed(b0d.at[pl.ds(off0, 16)], d16, mask=h0)
                off0 = off0 + plsc.all_reduce_population_count(h0)[0]
                plsc.store_compressed(b1s.at[pl.ds(off1, 16)], s16, mask=h1)
                plsc.store_compressed(b1d.at[pl.ds(off1, 16)], d16 - _HALF,
                                      mask=h1)
                off1 = off1 + plsc.all_reduce_population_count(h1)[0]
                plsc.addupdate_scatter(cnt_v, [d16], ones, mask=vm)
            return off0, off1

        stage(0, es0, ed0)

        def blk(t, carry):
            off0, off1 = carry
            bb0 = 2 * t
            wait_stage(es0, ed0)
            stage(bb0 + 1, es1, ed1)
            off0, off1 = process(bb0, es0, ed0, off0, off1)
            wait_stage(es1, ed1)
            stage(bb0 + 2, es0, ed0)
            off0, off1 = process(bb0 + 1, es1, ed1, off0, off1)
            return (off0, off1)

        off0, off1 = lax.fori_loop(
            0, 10, blk, (jnp.int32(0), jnp.int32(0)))
        wait_stage(es0, ed0)

        pltpu.async_copy(b0s.at[pl.ds(0, _SLOT)], psrc_hbm.at[wid, 0], osem)
        pltpu.async_copy(b1s.at[pl.ds(0, _SLOT)], psrc_hbm.at[wid, 1], osem)
        pltpu.async_copy(b0d.at[pl.ds(0, _SLOT)], pdoff_hbm.at[wid, 0], osem)
        pltpu.async_copy(b1d.at[pl.ds(0, _SLOT)], pdoff_hbm.at[wid, 1], osem)
        cout_v[0, pl.ds(0, 16)] = ziv + off0
        cout_v[1, pl.ds(0, 16)] = ziv + off1
        pltpu.async_copy(cout_v, counts_hbm.at[wid], osem)
        pltpu.async_copy(cnt_v, cnt_hbm.at[wid], osem)
        pltpu.make_async_copy(b0s.at[pl.ds(0, _SLOT)], psrc_hbm.at[wid, 0], osem).wait()
        pltpu.make_async_copy(b1s.at[pl.ds(0, _SLOT)], psrc_hbm.at[wid, 1], osem).wait()
        pltpu.make_async_copy(b0d.at[pl.ds(0, _SLOT)], pdoff_hbm.at[wid, 0], osem).wait()
        pltpu.make_async_copy(b1d.at[pl.ds(0, _SLOT)], pdoff_hbm.at[wid, 1], osem).wait()
        pltpu.make_async_copy(cout_v, counts_hbm.at[wid], osem).wait()
        pltpu.make_async_copy(cnt_v, cnt_hbm.at[wid], osem).wait()

    return k(srcE, dstE)


def _sc_agg(x0, psrc5, pdoff5, counts):
    mesh = plsc.VectorSubcoreMesh(
        core_axis_name="c", subcore_axis_name="s", num_cores=2, num_subcores=16
    )

    @functools.partial(
        pl.kernel,
        mesh=mesh,
        out_type=jax.ShapeDtypeStruct((2, _ACC, _D), jnp.float32),
        scratch_types=[
            pltpu.VMEM((4, 128), jnp.int32),
            pltpu.VMEM((4, 128), jnp.int32),
            pltpu.VMEM((32, _D), jnp.float32),
            pltpu.VMEM((32, _D), jnp.float32),
            pltpu.VMEM((16,), jnp.int32),
            pltpu.VMEM_SHARED((_N, _D), jnp.float32),
            pltpu.VMEM_SHARED((_ACC, _D), jnp.float32),
            pltpu.SemaphoreType.DMA,
            pltpu.SemaphoreType.DMA,
        ],
        compiler_params=pltpu.CompilerParams(needs_layout_passes=False),
    )
    def k(x0_hbm, psrc_hbm, pdoff_hbm, counts_hbm, agg_hbm,
          is_v, id_v, rows0_v, rows1_v, cnt_v, x0_s, acc_s, gsem, ssem):
        c = lax.axis_index("c")
        s = lax.axis_index("s")

        zfv = jnp.zeros((16,), jnp.float32)

        def zrow(i, carry):
            rows0_v[i, pl.ds(0, 16)] = zfv
            rows0_v[i, pl.ds(16, 16)] = zfv
            rows0_v[i, pl.ds(32, 16)] = zfv
            rows0_v[i, pl.ds(48, 16)] = zfv
            rows0_v[i, pl.ds(64, 16)] = zfv
            rows0_v[i, pl.ds(80, 16)] = zfv
            rows0_v[i, pl.ds(96, 16)] = zfv
            rows0_v[i, pl.ds(112, 16)] = zfv
            return carry

        lax.fori_loop(0, 32, zrow, 0)

        for t in range(10):
            pltpu.sync_copy(rows0_v, acc_s.at[pl.ds(s * 320 + t * 32, 32)])

        for kk in range(4):
            b = s + 16 * kk
            pltpu.sync_copy(x0_hbm.at[pl.ds(b * 128, 128)],
                            x0_s.at[pl.ds(b * 128, 128)])

        @pl.when(s <= 13)
        def _():
            b = s + 64
            pltpu.sync_copy(x0_hbm.at[pl.ds(b * 128, 128)],
                            x0_s.at[pl.ds(b * 128, 128)])

        @pl.when(s == 15)
        def _():
            pltpu.sync_copy(x0_hbm.at[pl.ds(9984, 16)],
                            x0_s.at[pl.ds(9984, 16)])

        plsc.subcore_barrier()

        bufs = (rows0_v, rows1_v)

        for u in range(2):
            w = 2 * s + u
            pltpu.sync_copy(counts_hbm.at[w, c], cnt_v)
            n = cnt_v[...][0]
            nb = lax.div(n + 511, 512)

            def blk(bb, carry):
                pltpu.sync_copy(psrc_hbm.at[w, c, bb], is_v)
                pltpu.sync_copy(pdoff_hbm.at[w, c, bb], id_v)

                def gfire(j, buf):
                    return pltpu.async_copy(
                        x0_s.at[is_v.at[j // 4, pl.ds((j % 4) * 32, 32)]],
                        buf, gsem)

                g = gfire(0, bufs[0])
                sprev = None
                for j in range(16):
                    cur = bufs[j % 2]
                    nxt = bufs[(j + 1) % 2]
                    g.wait()
                    if sprev is not None:
                        sprev.wait()
                    if j < 15:
                        g = gfire(j + 1, nxt)
                    sprev = pltpu.async_copy(
                        cur,
                        acc_s.at[id_v.at[j // 4, pl.ds((j % 4) * 32, 32)]],
                        ssem, add=True)
                sprev.wait()
                return carry

            lax.fori_loop(0, nb, blk, 0)

        plsc.subcore_barrier()
        pltpu.sync_copy(acc_s.at[pl.ds(s * 320, 320)],
                        agg_hbm.at[c, pl.ds(s * 320, 320)])

    return k(x0, psrc5, pdoff5, counts)


def _post_body(agg_ref, cnt_ref, rr_ref, xr_ref, hres_ref, wl_ref,
               wsc_ref, scal_ref, out_ref):
    cnt = jnp.sum(cnt_ref[...], axis=1, keepdims=True)
    aggm = agg_ref[...] * (1.0 / jnp.maximum(cnt, 1.0))
    h = jax.nn.relu(
        jnp.dot(aggm, wl_ref[...], preferred_element_type=jnp.float32)
        + xr_ref[...]
    )
    h = h + hres_ref[...]
    gnn = jnp.sum(h * wsc_ref[...], axis=1, keepdims=True)
    b_sc = scal_ref[0, 0]
    a = jax.nn.sigmoid(scal_ref[0, 1])
    out_ref[...] = a * rr_ref[...] + (1.0 - a) * (gnn + b_sc)


def _post(agg, cntT, rr, xr, hres, wl, wsc, scal):
    return pl.pallas_call(
        _post_body,
        grid=(_N // _BN,),
        in_specs=[
            pl.BlockSpec((_BN, _D), lambda i: (i, 0)),
            pl.BlockSpec((_BN, _NW), lambda i: (i, 0)),
            pl.BlockSpec((_BN, 1), lambda i: (i, 0)),
            pl.BlockSpec((_BN, _D), lambda i: (i, 0)),
            pl.BlockSpec((_BN, _D), lambda i: (i, 0)),
            pl.BlockSpec((_D, _D), lambda i: (0, 0)),
            pl.BlockSpec((1, _D), lambda i: (0, 0)),
            pl.BlockSpec(memory_space=pltpu.SMEM),
        ],
        out_specs=pl.BlockSpec((_BN, 1), lambda i: (i, 0)),
        out_shape=jax.ShapeDtypeStruct((_N, 1), jnp.float32),
    )(agg, cntT, rr, xr, hres, wl, wsc, scal)


def kernel(x, edge_index, reranker_scores, positions, lengths, W_fp, b_fp,
           W_l, b_l, W_r, W_res, b_res, W_sc, b_sc, alpha):
    x0, xr, hres = _pre(
        x,
        positions.reshape(_N, 1),
        lengths.reshape(_N, 1),
        W_fp[:_D],
        W_fp[_D:_D + 2],
        b_fp.reshape(1, _D),
        b_l.reshape(1, _D),
        W_r,
        W_res,
        b_res.reshape(1, _D),
    )

    srcE = edge_index[0].reshape(_E // 512, 4, 128)
    dstE = edge_index[1].reshape(_E // 512, 4, 128)

    psrc, pdoff, counts, cnt = _sc_partition(srcE, dstE)

    aggh = _sc_agg(
        x0,
        psrc.reshape(_NW, 2, _SLOT // 512, 4, 128),
        pdoff.reshape(_NW, 2, _SLOT // 512, 4, 128),
        counts,
    )
    agg = jnp.concatenate([aggh[0, :_HALF], aggh[1, :_HALF]], axis=0)

    scal = jnp.stack([b_sc[0], alpha]).reshape(1, 2)
    out = _post(
        agg,
        cnt[:, :_N].T,
        reranker_scores.reshape(_N, 1),
        xr,
        hres,
        W_l,
        W_sc.reshape(1, _D),
        scal,
    )
    return out.reshape(_N)


# speedup vs baseline: 1.0803x; 1.0205x over previous
import functools

import jax
import jax.numpy as jnp
from jax import lax
from jax.experimental import pallas as pl
from jax.experimental.pallas import tpu as pltpu
from jax.experimental.pallas import tpu_sc as plsc

_N = 10000
_D = 128
_E = 320000
_HALF = 5000
_ACC = 5120
_NC = 10016
_NW = 32
_EW = 10240
_EPAD = _NW * _EW
_SLOT = _EW
_BN = 400


def _pre_body(x_ref, pos_ref, len_ref, wa_ref, wpl_ref, b_ref,
              bl_ref, wr_ref, wres_ref, bres_ref, x0_ref, xr_ref, hres_ref):
    pos = pos_ref[...].astype(jnp.float32) * (1.0 / 50.0)
    ln = len_ref[...].astype(jnp.float32) * (1.0 / 500.0)
    x0 = jnp.dot(x_ref[...], wa_ref[...], preferred_element_type=jnp.float32)
    x0 = x0 + pos * wpl_ref[0:1, :] + ln * wpl_ref[1:2, :] + b_ref[...]
    x0_ref[...] = x0
    xr_ref[...] = (
        jnp.dot(x0, wr_ref[...], preferred_element_type=jnp.float32)
        + bl_ref[...])
    hres_ref[...] = (
        jnp.dot(x0, wres_ref[...], preferred_element_type=jnp.float32)
        + bres_ref[...])


def _pre(x, pos, ln, wa, wpl, b, bl, wr, wres, bres):
    return pl.pallas_call(
        _pre_body,
        grid=(_N // _BN,),
        in_specs=[
            pl.BlockSpec((_BN, _D), lambda i: (i, 0)),
            pl.BlockSpec((_BN, 1), lambda i: (i, 0)),
            pl.BlockSpec((_BN, 1), lambda i: (i, 0)),
            pl.BlockSpec((_D, _D), lambda i: (0, 0)),
            pl.BlockSpec((2, _D), lambda i: (0, 0)),
            pl.BlockSpec((1, _D), lambda i: (0, 0)),
            pl.BlockSpec((1, _D), lambda i: (0, 0)),
            pl.BlockSpec((_D, _D), lambda i: (0, 0)),
            pl.BlockSpec((_D, _D), lambda i: (0, 0)),
            pl.BlockSpec((1, _D), lambda i: (0, 0)),
        ],
        out_specs=[
            pl.BlockSpec((_BN, _D), lambda i: (i, 0)),
            pl.BlockSpec((_BN, _D), lambda i: (i, 0)),
            pl.BlockSpec((_BN, _D), lambda i: (i, 0)),
        ],
        out_shape=[
            jax.ShapeDtypeStruct((_N, _D), jnp.float32),
            jax.ShapeDtypeStruct((_N, _D), jnp.float32),
            jax.ShapeDtypeStruct((_N, _D), jnp.float32),
        ],
    )(x, pos, ln, wa, wpl, b, bl, wr, wres, bres)


def _sc_partition(srcE, dstE):
    mesh = plsc.VectorSubcoreMesh(
        core_axis_name="c", subcore_axis_name="s", num_cores=2, num_subcores=16
    )

    @functools.partial(
        pl.kernel,
        mesh=mesh,
        out_type=[
            jax.ShapeDtypeStruct((_NW, 2, _SLOT), jnp.int32),
            jax.ShapeDtypeStruct((_NW, 2, _SLOT), jnp.int32),
            jax.ShapeDtypeStruct((_NW, 2, 16), jnp.int32),
            jax.ShapeDtypeStruct((_NW, _NC), jnp.float32),
        ],
        scratch_types=[
            pltpu.VMEM((4, 128), jnp.int32),
            pltpu.VMEM((4, 128), jnp.int32),
            pltpu.VMEM((4, 128), jnp.int32),
            pltpu.VMEM((4, 128), jnp.int32),
            pltpu.VMEM((_SLOT + 16,), jnp.int32),
            pltpu.VMEM((_SLOT + 16,), jnp.int32),
            pltpu.VMEM((_SLOT + 16,), jnp.int32),
            pltpu.VMEM((_SLOT + 16,), jnp.int32),
            pltpu.VMEM((_NC,), jnp.float32),
            pltpu.VMEM((2, 16), jnp.int32),
            pltpu.SemaphoreType.DMA,
            pltpu.SemaphoreType.DMA,
        ],
        compiler_params=pltpu.CompilerParams(needs_layout_passes=False),
    )
    def k(srcE_hbm, dstE_hbm, psrc_hbm, pdoff_hbm, counts_hbm, cnt_hbm,
          es0, ed0, es1, ed1, b0s, b0d, b1s, b1d, cnt_v, cout_v,
          stsem, osem):
        c = lax.axis_index("c")
        s = lax.axis_index("s")
        wid = c * 16 + s
        nblk = jnp.where(wid < 17, 20, 19)
        start = wid * 19 + jnp.minimum(wid, 17)

        ziv = jnp.zeros((16,), jnp.int32)
        zfv = jnp.zeros((16,), jnp.float32)
        ones = jnp.full((16,), 1.0, jnp.float32)
        dummy = ziv + (5008 + lax.div(wid, 2))

        def pre(i, carry):
            b0s[pl.ds(i * 16, 16)] = ziv
            b0d[pl.ds(i * 16, 16)] = dummy
            b1s[pl.ds(i * 16, 16)] = ziv
            b1d[pl.ds(i * 16, 16)] = dummy
            return carry

        lax.fori_loop(0, (_SLOT + 16) // 16, pre, 0)

        def zcnt(i, carry):
            cnt_v[pl.ds(i * 16, 16)] = zfv
            return carry

        lax.fori_loop(0, _NC // 16, zcnt, 0)

        nlast = _E // 512 - 1

        def stage(bidx, sbuf, dbuf):
            bi = jnp.minimum(start + bidx, nlast)
            pltpu.async_copy(srcE_hbm.at[bi], sbuf, stsem)
            pltpu.async_copy(dstE_hbm.at[bi], dbuf, stsem)

        def wait_stage(sbuf, dbuf):
            pltpu.make_async_copy(srcE_hbm.at[0], sbuf, stsem).wait()
            pltpu.make_async_copy(dstE_hbm.at[0], dbuf, stsem).wait()

        def process(bb, es_v, ed_v, off0, off1):
            vm = (ziv + bb) < (ziv + nblk)
            for g in range(32):
                r, co = g // 8, (g % 8) * 16
                s16 = es_v[r, pl.ds(co, 16)]
                d16 = ed_v[r, pl.ds(co, 16)]
                h0 = jnp.logical_and(d16 < _HALF, vm)
                h1 = jnp.logical_and(d16 >= _HALF, vm)
                plsc.store_compressed(b0s.at[pl.ds(off0, 16)], s16, mask=h0)
                plsc.store_compressed(b0d.at[pl.ds(off0, 16)], d16, mask=h0)
                off0 = off0 + plsc.all_reduce_population_count(h0)[0]
                plsc.store_compressed(b1s.at[pl.ds(off1, 16)], s16, mask=h1)
                plsc.store_compressed(b1d.at[pl.ds(off1, 16)], d16 - _HALF,
                                      mask=h1)
                off1 = off1 + plsc.all_reduce_population_count(h1)[0]
                plsc.addupdate_scatter(cnt_v, [d16], ones, mask=vm)
            return off0, off1

        stage(0, es0, ed0)

        def blk(t, carry):
            off0, off1 = carry
            bb0 = 2 * t
            wait_stage(es0, ed0)
            stage(bb0 + 1, es1, ed1)
            off0, off1 = process(bb0, es0, ed0, off0, off1)
            wait_stage(es1, ed1)
            stage(bb0 + 2, es0, ed0)
            off0, off1 = process(bb0 + 1, es1, ed1, off0, off1)
            return (off0, off1)

        off0, off1 = lax.fori_loop(
            0, 10, blk, (jnp.int32(0), jnp.int32(0)))
        wait_stage(es0, ed0)

        pltpu.async_copy(b0s.at[pl.ds(0, _SLOT)], psrc_hbm.at[wid, 0], osem)
        pltpu.async_copy(b1s.at[pl.ds(0, _SLOT)], psrc_hbm.at[wid, 1], osem)
        pltpu.async_copy(b0d.at[pl.ds(0, _SLOT)], pdoff_hbm.at[wid, 0], osem)
        pltpu.async_copy(b1d.at[pl.ds(0, _SLOT)], pdoff_hbm.at[wid, 1], osem)
        cout_v[0, pl.ds(0, 16)] = ziv + off0
        cout_v[1, pl.ds(0, 16)] = ziv + off1
        pltpu.async_copy(cout_v, counts_hbm.at[wid], osem)
        pltpu.async_copy(cnt_v, cnt_hbm.at[wid], osem)
        pltpu.make_async_copy(b0s.at[pl.ds(0, _SLOT)], psrc_hbm.at[wid, 0], osem).wait()
        pltpu.make_async_copy(b1s.at[pl.ds(0, _SLOT)], psrc_hbm.at[wid, 1], osem).wait()
        pltpu.make_async_copy(b0d.at[pl.ds(0, _SLOT)], pdoff_hbm.at[wid, 0], osem).wait()
        pltpu.make_async_copy(b1d.at[pl.ds(0, _SLOT)], pdoff_hbm.at[wid, 1], osem).wait()
        pltpu.make_async_copy(cout_v, counts_hbm.at[wid], osem).wait()
        pltpu.make_async_copy(cnt_v, cnt_hbm.at[wid], osem).wait()

    return k(srcE, dstE)


def _sc_agg(x0, psrc5, pdoff5, counts):
    mesh = plsc.VectorSubcoreMesh(
        core_axis_name="c", subcore_axis_name="s", num_cores=2, num_subcores=16
    )

    @functools.partial(
        pl.kernel,
        mesh=mesh,
        out_type=jax.ShapeDtypeStruct((2, _ACC, _D), jnp.float32),
        scratch_types=[
            pltpu.VMEM((4, 128), jnp.int32),
            pltpu.VMEM((4, 128), jnp.int32),
            pltpu.VMEM((32, _D), jnp.float32),
            pltpu.VMEM((32, _D), jnp.float32),
            pltpu.VMEM((16,), jnp.int32),
            pltpu.VMEM_SHARED((_N, _D), jnp.float32),
            pltpu.VMEM_SHARED((_ACC, _D), jnp.float32),
            pltpu.SemaphoreType.DMA,
            pltpu.SemaphoreType.DMA,
        ],
        compiler_params=pltpu.CompilerParams(needs_layout_passes=False),
    )
    def k(x0_hbm, psrc_hbm, pdoff_hbm, counts_hbm, agg_hbm,
          is_v, id_v, rows0_v, rows1_v, cnt_v, x0_s, acc_s, gsem, ssem):
        c = lax.axis_index("c")
        s = lax.axis_index("s")

        zfv = jnp.zeros((16,), jnp.float32)

        def zrow(i, carry):
            rows0_v[i, pl.ds(0, 16)] = zfv
            rows0_v[i, pl.ds(16, 16)] = zfv
            rows0_v[i, pl.ds(32, 16)] = zfv
            rows0_v[i, pl.ds(48, 16)] = zfv
            rows0_v[i, pl.ds(64, 16)] = zfv
            rows0_v[i, pl.ds(80, 16)] = zfv
            rows0_v[i, pl.ds(96, 16)] = zfv
            rows0_v[i, pl.ds(112, 16)] = zfv
            return carry

        lax.fori_loop(0, 32, zrow, 0)

        for t in range(10):
            pltpu.sync_copy(rows0_v, acc_s.at[pl.ds(s * 320 + t * 32, 32)])

        for kk in range(4):
            b = s + 16 * kk
            pltpu.sync_copy(x0_hbm.at[pl.ds(b * 128, 128)],
                            x0_s.at[pl.ds(b * 128, 128)])

        @pl.when(s <= 13)
        def _():
            b = s + 64
            pltpu.sync_copy(x0_hbm.at[pl.ds(b * 128, 128)],
                            x0_s.at[pl.ds(b * 128, 128)])

        @pl.when(s == 15)
        def _():
            pltpu.sync_copy(x0_hbm.at[pl.ds(9984, 16)],
                            x0_s.at[pl.ds(9984, 16)])

        plsc.subcore_barrier()

        bufs = (rows0_v, rows1_v)

        for u in range(2):
            w = 2 * s + u
            pltpu.sync_copy(counts_hbm.at[w, c], cnt_v)
            n = cnt_v[...][0]
            nb = lax.div(n + 511, 512)
            nc = lax.div(n + 31, 32)

            def gfire(j, buf):
                return pltpu.async_copy(
                    x0_s.at[is_v.at[j // 4, pl.ds((j % 4) * 32, 32)]],
                    buf, gsem)

            def sfire(j, buf):
                return pltpu.async_copy(
                    buf,
                    acc_s.at[id_v.at[j // 4, pl.ds((j % 4) * 32, 32)]],
                    ssem, add=True)

            def blk(bb, carry):
                pltpu.sync_copy(psrc_hbm.at[w, c, bb], is_v)
                pltpu.sync_copy(pdoff_hbm.at[w, c, bb], id_v)
                g = gfire(0, bufs[0])
                sprev = None
                for j in range(16):
                    cur = bufs[j % 2]
                    nxt = bufs[(j + 1) % 2]
                    g.wait()
                    if sprev is not None:
                        sprev.wait()
                    if j < 15:
                        g = gfire(j + 1, nxt)
                    sprev = sfire(j, cur)
                sprev.wait()
                return carry

            lax.fori_loop(0, jnp.maximum(nb - 1, 0), blk, 0)

            @pl.when(n > 0)
            def _():
                bb = nb - 1
                pltpu.sync_copy(psrc_hbm.at[w, c, bb], is_v)
                pltpu.sync_copy(pdoff_hbm.at[w, c, bb], id_v)
                npair = lax.div(nc - bb * 16 + 1, 2)

                def pair(t, carry):
                    j0 = 2 * t
                    g0 = gfire(j0, bufs[0])
                    g1 = gfire(j0 + 1, bufs[1])
                    g0.wait()
                    s0 = sfire(j0, bufs[0])
                    g1.wait()
                    s0.wait()
                    s1 = sfire(j0 + 1, bufs[1])
                    s1.wait()
                    return carry

                lax.fori_loop(0, npair, pair, 0)

        plsc.subcore_barrier()
        pltpu.sync_copy(acc_s.at[pl.ds(s * 320, 320)],
                        agg_hbm.at[c, pl.ds(s * 320, 320)])

    return k(x0, psrc5, pdoff5, counts)


def _post_body(agg_ref, cnt_ref, rr_ref, xr_ref, hres_ref, wl_ref,
               wsc_ref, scal_ref, out_ref):
    cnt = jnp.sum(cnt_ref[...], axis=1, keepdims=True)
    aggm = agg_ref[...] * (1.0 / jnp.maximum(cnt, 1.0))
    h = jax.nn.relu(
        jnp.dot(aggm, wl_ref[...], preferred_element_type=jnp.float32)
        + xr_ref[...]
    )
    h = h + hres_ref[...]
    gnn = jnp.sum(h * wsc_ref[...], axis=1, keepdims=True)
    b_sc = scal_ref[0, 0]
    a = jax.nn.sigmoid(scal_ref[0, 1])
    out_ref[...] = a * rr_ref[...] + (1.0 - a) * (gnn + b_sc)


def _post(agg, cntT, rr, xr, hres, wl, wsc, scal):
    return pl.pallas_call(
        _post_body,
        grid=(_N // _BN,),
        in_specs=[
            pl.BlockSpec((_BN, _D), lambda i: (i, 0)),
            pl.BlockSpec((_BN, _NW), lambda i: (i, 0)),
            pl.BlockSpec((_BN, 1), lambda i: (i, 0)),
            pl.BlockSpec((_BN, _D), lambda i: (i, 0)),
            pl.BlockSpec((_BN, _D), lambda i: (i, 0)),
            pl.BlockSpec((_D, _D), lambda i: (0, 0)),
            pl.BlockSpec((1, _D), lambda i: (0, 0)),
            pl.BlockSpec(memory_space=pltpu.SMEM),
        ],
        out_specs=pl.BlockSpec((_BN, 1), lambda i: (i, 0)),
        out_shape=jax.ShapeDtypeStruct((_N, 1), jnp.float32),
    )(agg, cntT, rr, xr, hres, wl, wsc, scal)


def kernel(x, edge_index, reranker_scores, positions, lengths, W_fp, b_fp,
           W_l, b_l, W_r, W_res, b_res, W_sc, b_sc, alpha):
    x0, xr, hres = _pre(
        x,
        positions.reshape(_N, 1),
        lengths.reshape(_N, 1),
        W_fp[:_D],
        W_fp[_D:_D + 2],
        b_fp.reshape(1, _D),
        b_l.reshape(1, _D),
        W_r,
        W_res,
        b_res.reshape(1, _D),
    )

    srcE = edge_index[0].reshape(_E // 512, 4, 128)
    dstE = edge_index[1].reshape(_E // 512, 4, 128)

    psrc, pdoff, counts, cnt = _sc_partition(srcE, dstE)

    aggh = _sc_agg(
        x0,
        psrc.reshape(_NW, 2, _SLOT // 512, 4, 128),
        pdoff.reshape(_NW, 2, _SLOT // 512, 4, 128),
        counts,
    )
    agg = jnp.concatenate([aggh[0, :_HALF], aggh[1, :_HALF]], axis=0)

    scal = jnp.stack([b_sc[0], alpha]).reshape(1, 2)
    out = _post(
        agg,
        cnt[:, :_N].T,
        reranker_scores.reshape(_N, 1),
        xr,
        hres,
        W_l,
        W_sc.reshape(1, _D),
        scal,
    )
    return out.reshape(_N)


# speedup vs baseline: 1.1090x; 1.0266x over previous
import functools

import jax
import jax.numpy as jnp
from jax import lax
from jax.experimental import pallas as pl
from jax.experimental.pallas import tpu as pltpu
from jax.experimental.pallas import tpu_sc as plsc

_N = 10000
_D = 128
_E = 320000
_HALF = 5000
_ACC = 5120
_NC = 10016
_NW = 32
_EW = 10240
_EPAD = _NW * _EW
_SLOT = _EW
_BN = 400


def _pre_body(x_ref, pos_ref, len_ref, wa_ref, wpl_ref, b_ref,
              bl_ref, wr_ref, wres_ref, bres_ref, x0_ref, xr_ref, hres_ref):
    pos = pos_ref[...].astype(jnp.float32) * (1.0 / 50.0)
    ln = len_ref[...].astype(jnp.float32) * (1.0 / 500.0)
    x0 = jnp.dot(x_ref[...], wa_ref[...], preferred_element_type=jnp.float32)
    x0 = x0 + pos * wpl_ref[0:1, :] + ln * wpl_ref[1:2, :] + b_ref[...]
    x0_ref[...] = x0
    xr_ref[...] = (
        jnp.dot(x0, wr_ref[...], preferred_element_type=jnp.float32)
        + bl_ref[...])
    hres_ref[...] = (
        jnp.dot(x0, wres_ref[...], preferred_element_type=jnp.float32)
        + bres_ref[...])


def _pre(x, pos, ln, wa, wpl, b, bl, wr, wres, bres):
    return pl.pallas_call(
        _pre_body,
        grid=(_N // _BN,),
        in_specs=[
            pl.BlockSpec((_BN, _D), lambda i: (i, 0)),
            pl.BlockSpec((_BN, 1), lambda i: (i, 0)),
            pl.BlockSpec((_BN, 1), lambda i: (i, 0)),
            pl.BlockSpec((_D, _D), lambda i: (0, 0)),
            pl.BlockSpec((2, _D), lambda i: (0, 0)),
            pl.BlockSpec((1, _D), lambda i: (0, 0)),
            pl.BlockSpec((1, _D), lambda i: (0, 0)),
            pl.BlockSpec((_D, _D), lambda i: (0, 0)),
            pl.BlockSpec((_D, _D), lambda i: (0, 0)),
            pl.BlockSpec((1, _D), lambda i: (0, 0)),
        ],
        out_specs=[
            pl.BlockSpec((_BN, _D), lambda i: (i, 0)),
            pl.BlockSpec((_BN, _D), lambda i: (i, 0)),
            pl.BlockSpec((_BN, _D), lambda i: (i, 0)),
        ],
        out_shape=[
            jax.ShapeDtypeStruct((_N, _D), jnp.float32),
            jax.ShapeDtypeStruct((_N, _D), jnp.float32),
            jax.ShapeDtypeStruct((_N, _D), jnp.float32),
        ],
    )(x, pos, ln, wa, wpl, b, bl, wr, wres, bres)


def _sc_partition(srcE, dstE):
    mesh = plsc.VectorSubcoreMesh(
        core_axis_name="c", subcore_axis_name="s", num_cores=2, num_subcores=16
    )

    @functools.partial(
        pl.kernel,
        mesh=mesh,
        out_type=[
            jax.ShapeDtypeStruct((_NW, 2, _SLOT), jnp.int32),
            jax.ShapeDtypeStruct((_NW, 2, _SLOT), jnp.int32),
            jax.ShapeDtypeStruct((_NW, 2, 16), jnp.int32),
            jax.ShapeDtypeStruct((_NW, _NC), jnp.float32),
        ],
        scratch_types=[
            pltpu.VMEM((4, 128), jnp.int32),
            pltpu.VMEM((4, 128), jnp.int32),
            pltpu.VMEM((4, 128), jnp.int32),
            pltpu.VMEM((4, 128), jnp.int32),
            pltpu.VMEM((_SLOT + 16,), jnp.int32),
            pltpu.VMEM((_SLOT + 16,), jnp.int32),
            pltpu.VMEM((_SLOT + 16,), jnp.int32),
            pltpu.VMEM((_SLOT + 16,), jnp.int32),
            pltpu.VMEM((_NC,), jnp.float32),
            pltpu.VMEM((2, 16), jnp.int32),
            pltpu.SemaphoreType.DMA,
            pltpu.SemaphoreType.DMA,
        ],
        compiler_params=pltpu.CompilerParams(needs_layout_passes=False),
    )
    def k(srcE_hbm, dstE_hbm, psrc_hbm, pdoff_hbm, counts_hbm, cnt_hbm,
          es0, ed0, es1, ed1, b0s, b0d, b1s, b1d, cnt_v, cout_v,
          stsem, osem):
        c = lax.axis_index("c")
        s = lax.axis_index("s")
        wid = c * 16 + s
        nblk = jnp.where(wid < 17, 20, 19)
        start = wid * 19 + jnp.minimum(wid, 17)

        ziv = jnp.zeros((16,), jnp.int32)
        zfv = jnp.zeros((16,), jnp.float32)
        ones = jnp.full((16,), 1.0, jnp.float32)
        dummy = ziv + (5008 + lax.div(wid, 2))

        def pre(i, carry):
            b0s[pl.ds(i * 16, 16)] = ziv
            b0d[pl.ds(i * 16, 16)] = dummy
            b1s[pl.ds(i * 16, 16)] = ziv
            b1d[pl.ds(i * 16, 16)] = dummy
            return carry

        lax.fori_loop(0, (_SLOT + 16) // 16, pre, 0)

        def zcnt(i, carry):
            cnt_v[pl.ds(i * 16, 16)] = zfv
            return carry

        lax.fori_loop(0, _NC // 16, zcnt, 0)

        nlast = _E // 512 - 1

        def stage(bidx, sbuf, dbuf):
            bi = jnp.minimum(start + bidx, nlast)
            pltpu.async_copy(srcE_hbm.at[bi], sbuf, stsem)
            pltpu.async_copy(dstE_hbm.at[bi], dbuf, stsem)

        def wait_stage(sbuf, dbuf):
            pltpu.make_async_copy(srcE_hbm.at[0], sbuf, stsem).wait()
            pltpu.make_async_copy(dstE_hbm.at[0], dbuf, stsem).wait()

        def process(bb, es_v, ed_v, off0, off1):
            vm = (ziv + bb) < (ziv + nblk)
            for g in range(32):
                r, co = g // 8, (g % 8) * 16
                s16 = es_v[r, pl.ds(co, 16)]
                d16 = ed_v[r, pl.ds(co, 16)]
                h0 = jnp.logical_and(d16 < _HALF, vm)
                h1 = jnp.logical_and(d16 >= _HALF, vm)
                plsc.store_compressed(b0s.at[pl.ds(off0, 16)], s16, mask=h0)
                plsc.store_compressed(b0d.at[pl.ds(off0, 16)], d16, mask=h0)
                off0 = off0 + plsc.all_reduce_population_count(h0)[0]
                plsc.store_compressed(b1s.at[pl.ds(off1, 16)], s16, mask=h1)
                plsc.store_compressed(b1d.at[pl.ds(off1, 16)], d16 - _HALF,
                                      mask=h1)
                off1 = off1 + plsc.all_reduce_population_count(h1)[0]
                plsc.addupdate_scatter(cnt_v, [d16], ones, mask=vm)
            return off0, off1

        stage(0, es0, ed0)

        def blk(t, carry):
            off0, off1 = carry
            bb0 = 2 * t
            wait_stage(es0, ed0)
            stage(bb0 + 1, es1, ed1)
            off0, off1 = process(bb0, es0, ed0, off0, off1)
            wait_stage(es1, ed1)
            stage(bb0 + 2, es0, ed0)
            off0, off1 = process(bb0 + 1, es1, ed1, off0, off1)
            return (off0, off1)

        off0, off1 = lax.fori_loop(
            0, 10, blk, (jnp.int32(0), jnp.int32(0)))
        wait_stage(es0, ed0)

        pltpu.async_copy(b0s.at[pl.ds(0, _SLOT)], psrc_hbm.at[wid, 0], osem)
        pltpu.async_copy(b1s.at[pl.ds(0, _SLOT)], psrc_hbm.at[wid, 1], osem)
        pltpu.async_copy(b0d.at[pl.ds(0, _SLOT)], pdoff_hbm.at[wid, 0], osem)
        pltpu.async_copy(b1d.at[pl.ds(0, _SLOT)], pdoff_hbm.at[wid, 1], osem)
        cout_v[0, pl.ds(0, 16)] = ziv + off0
        cout_v[1, pl.ds(0, 16)] = ziv + off1
        pltpu.async_copy(cout_v, counts_hbm.at[wid], osem)
        pltpu.async_copy(cnt_v, cnt_hbm.at[wid], osem)
        pltpu.make_async_copy(b0s.at[pl.ds(0, _SLOT)], psrc_hbm.at[wid, 0], osem).wait()
        pltpu.make_async_copy(b1s.at[pl.ds(0, _SLOT)], psrc_hbm.at[wid, 1], osem).wait()
        pltpu.make_async_copy(b0d.at[pl.ds(0, _SLOT)], pdoff_hbm.at[wid, 0], osem).wait()
        pltpu.make_async_copy(b1d.at[pl.ds(0, _SLOT)], pdoff_hbm.at[wid, 1], osem).wait()
        pltpu.make_async_copy(cout_v, counts_hbm.at[wid], osem).wait()
        pltpu.make_async_copy(cnt_v, cnt_hbm.at[wid], osem).wait()

    return k(srcE, dstE)


def _sc_agg(x0, psrc5, pdoff5, counts):
    mesh = plsc.VectorSubcoreMesh(
        core_axis_name="c", subcore_axis_name="s", num_cores=2, num_subcores=16
    )

    @functools.partial(
        pl.kernel,
        mesh=mesh,
        out_type=jax.ShapeDtypeStruct((_N, _D), jnp.float32),
        scratch_types=[
            pltpu.VMEM((4, 128), jnp.int32),
            pltpu.VMEM((4, 128), jnp.int32),
            pltpu.VMEM((32, _D), jnp.float32),
            pltpu.VMEM((32, _D), jnp.float32),
            pltpu.VMEM((2, 16), jnp.int32),
            pltpu.VMEM_SHARED((_N, _D), jnp.float32),
            pltpu.VMEM_SHARED((_ACC, _D), jnp.float32),
            pltpu.SemaphoreType.DMA,
            pltpu.SemaphoreType.DMA,
        ],
        compiler_params=pltpu.CompilerParams(needs_layout_passes=False),
    )
    def k(x0_hbm, psrc_hbm, pdoff_hbm, counts_hbm, agg_hbm,
          is_v, id_v, rows0_v, rows1_v, cnt_v, x0_s, acc_s, gsem, ssem):
        c = lax.axis_index("c")
        s = lax.axis_index("s")

        zfv = jnp.zeros((16,), jnp.float32)

        def zrow(i, carry):
            rows0_v[i, pl.ds(0, 16)] = zfv
            rows0_v[i, pl.ds(16, 16)] = zfv
            rows0_v[i, pl.ds(32, 16)] = zfv
            rows0_v[i, pl.ds(48, 16)] = zfv
            rows0_v[i, pl.ds(64, 16)] = zfv
            rows0_v[i, pl.ds(80, 16)] = zfv
            rows0_v[i, pl.ds(96, 16)] = zfv
            rows0_v[i, pl.ds(112, 16)] = zfv
            return carry

        lax.fori_loop(0, 32, zrow, 0)

        for t in range(10):
            pltpu.sync_copy(rows0_v, acc_s.at[pl.ds(s * 320 + t * 32, 32)])

        for kk in range(4):
            b = s + 16 * kk
            pltpu.sync_copy(x0_hbm.at[pl.ds(b * 128, 128)],
                            x0_s.at[pl.ds(b * 128, 128)])

        @pl.when(s <= 13)
        def _():
            b = s + 64
            pltpu.sync_copy(x0_hbm.at[pl.ds(b * 128, 128)],
                            x0_s.at[pl.ds(b * 128, 128)])

        @pl.when(s == 15)
        def _():
            pltpu.sync_copy(x0_hbm.at[pl.ds(9984, 16)],
                            x0_s.at[pl.ds(9984, 16)])

        pltpu.sync_copy(counts_hbm.at[2 * s, c], cnt_v.at[0])
        pltpu.sync_copy(counts_hbm.at[2 * s + 1, c], cnt_v.at[1])

        plsc.subcore_barrier()

        bufs = (rows0_v, rows1_v)

        for u in range(2):
            w = 2 * s + u
            n = cnt_v[u, pl.ds(0, 16)][0]
            nb = lax.div(n + 511, 512)
            nc = lax.div(n + 31, 32)

            def gfire(j, buf):
                return pltpu.async_copy(
                    x0_s.at[is_v.at[j // 4, pl.ds((j % 4) * 32, 32)]],
                    buf, gsem)

            def sfire(j, buf):
                return pltpu.async_copy(
                    buf,
                    acc_s.at[id_v.at[j // 4, pl.ds((j % 4) * 32, 32)]],
                    ssem, add=True)

            def blk(bb, carry):
                pltpu.sync_copy(psrc_hbm.at[w, c, bb], is_v)
                pltpu.sync_copy(pdoff_hbm.at[w, c, bb], id_v)
                g = gfire(0, bufs[0])
                sprev = None
                for j in range(16):
                    cur = bufs[j % 2]
                    nxt = bufs[(j + 1) % 2]
                    g.wait()
                    if sprev is not None:
                        sprev.wait()
                    if j < 15:
                        g = gfire(j + 1, nxt)
                    sprev = sfire(j, cur)
                sprev.wait()
                return carry

            lax.fori_loop(0, jnp.maximum(nb - 1, 0), blk, 0)

            @pl.when(n > 0)
            def _():
                bb = nb - 1
                pltpu.sync_copy(psrc_hbm.at[w, c, bb], is_v)
                pltpu.sync_copy(pdoff_hbm.at[w, c, bb], id_v)
                npair = lax.div(nc - bb * 16 + 1, 2)

                def pair(t, carry):
                    j0 = 2 * t
                    g0 = gfire(j0, bufs[0])
                    g1 = gfire(j0 + 1, bufs[1])
                    g0.wait()
                    s0 = sfire(j0, bufs[0])
                    g1.wait()
                    s0.wait()
                    s1 = sfire(j0 + 1, bufs[1])
                    s1.wait()
                    return carry

                lax.fori_loop(0, npair, pair, 0)

        plsc.subcore_barrier()

        @pl.when(s < 15)
        def _():
            pltpu.sync_copy(acc_s.at[pl.ds(s * 320, 320)],
                            agg_hbm.at[pl.ds(c * _HALF + s * 320, 320)])

        @pl.when(s == 15)
        def _():
            pltpu.sync_copy(acc_s.at[pl.ds(4800, 200)],
                            agg_hbm.at[pl.ds(c * _HALF + 4800, 200)])

    return k(x0, psrc5, pdoff5, counts)


def _post_body(agg_ref, cnt_ref, rr_ref, xr_ref, hres_ref, wl_ref,
               wsc_ref, scal_ref, out_ref):
    cnt = jnp.sum(cnt_ref[...], axis=1, keepdims=True)
    aggm = agg_ref[...] * (1.0 / jnp.maximum(cnt, 1.0))
    h = jax.nn.relu(
        jnp.dot(aggm, wl_ref[...], preferred_element_type=jnp.float32)
        + xr_ref[...]
    )
    h = h + hres_ref[...]
    gnn = jnp.sum(h * wsc_ref[...], axis=1, keepdims=True)
    b_sc = scal_ref[0, 0]
    a = jax.nn.sigmoid(scal_ref[0, 1])
    out_ref[...] = a * rr_ref[...] + (1.0 - a) * (gnn + b_sc)


def _post(agg, cntT, rr, xr, hres, wl, wsc, scal):
    return pl.pallas_call(
        _post_body,
        grid=(_N // _BN,),
        in_specs=[
            pl.BlockSpec((_BN, _D), lambda i: (i, 0)),
            pl.BlockSpec((_BN, _NW), lambda i: (i, 0)),
            pl.BlockSpec((_BN, 1), lambda i: (i, 0)),
            pl.BlockSpec((_BN, _D), lambda i: (i, 0)),
            pl.BlockSpec((_BN, _D), lambda i: (i, 0)),
            pl.BlockSpec((_D, _D), lambda i: (0, 0)),
            pl.BlockSpec((1, _D), lambda i: (0, 0)),
            pl.BlockSpec(memory_space=pltpu.SMEM),
        ],
        out_specs=pl.BlockSpec((_BN, 1), lambda i: (i, 0)),
        out_shape=jax.ShapeDtypeStruct((_N, 1), jnp.float32),
    )(agg, cntT, rr, xr, hres, wl, wsc, scal)


def kernel(x, edge_index, reranker_scores, positions, lengths, W_fp, b_fp,
           W_l, b_l, W_r, W_res, b_res, W_sc, b_sc, alpha):
    x0, xr, hres = _pre(
        x,
        positions.reshape(_N, 1),
        lengths.reshape(_N, 1),
        W_fp[:_D],
        W_fp[_D:_D + 2],
        b_fp.reshape(1, _D),
        b_l.reshape(1, _D),
        W_r,
        W_res,
        b_res.reshape(1, _D),
    )

    srcE = edge_index[0].reshape(_E // 512, 4, 128)
    dstE = edge_index[1].reshape(_E // 512, 4, 128)

    psrc, pdoff, counts, cnt = _sc_partition(srcE, dstE)

    agg = _sc_agg(
        x0,
        psrc.reshape(_NW, 2, _SLOT // 512, 4, 128),
        pdoff.reshape(_NW, 2, _SLOT // 512, 4, 128),
        counts,
    )

    scal = jnp.stack([b_sc[0], alpha]).reshape(1, 2)
    out = _post(
        agg,
        cnt[:, :_N].T,
        reranker_scores.reshape(_N, 1),
        xr,
        hres,
        W_l,
        W_sc.reshape(1, _D),
        scal,
    )
    return out.reshape(_N)
